# baseline (speedup 1.0000x reference)
import functools

import jax
import jax.numpy as jnp
from jax import lax
from jax.experimental import pallas as pl
from jax.experimental.pallas import tpu as pltpu
from jax.experimental.pallas import tpu_sc as plsc

N_FIELDS = 26
VOCAB = 100000
EMBD = 32
BATCH = 16384

NUM_CORES = 2
NUM_SUBCORES = 16
NW = NUM_CORES * NUM_SUBCORES
BPW = BATCH // NW
CHUNK = 128
NCH = BPW // CHUNK


def _sc_embed(tf_t, tabs):
    mesh = plsc.VectorSubcoreMesh(core_axis_name="c", subcore_axis_name="s")

    @functools.partial(
        pl.kernel,
        mesh=mesh,
        out_type=jax.ShapeDtypeStruct((N_FIELDS, BATCH, EMBD), jnp.float32),
        compiler_params=pltpu.CompilerParams(use_tc_tiling_on_sc=False),
        scratch_types=[
            pltpu.VMEM((BPW,), jnp.int32),
            pltpu.VMEM((2, CHUNK, EMBD), jnp.float32),
            pltpu.SemaphoreType.DMA,
            pltpu.SemaphoreType.DMA,
            pltpu.SemaphoreType.DMA,
        ],
    )
    def k(tf_hbm, *refs):
        tab_hbms = refs[:N_FIELDS]
        out_hbm = refs[N_FIELDS]
        idx_v, ring_v, isem, gsem, wsem = refs[N_FIELDS + 1:]
        wid = lax.axis_index("s") * NUM_CORES + lax.axis_index("c")
        b0 = wid * BPW

        for f in range(N_FIELDS):
            pltpu.async_copy(
                tf_hbm.at[f, pl.ds(b0, BPW)], idx_v, isem
            ).wait()

            def step_body(q, carry, _tab=tab_hbms[f], _f=f):
                par = lax.rem(q, 2)

                @pl.when(q >= 2)
                def _():
                    pltpu.make_async_copy(
                        ring_v.at[par],
                        out_hbm.at[_f, pl.ds(b0, CHUNK)],
                        wsem,
                    ).wait()

                pltpu.async_copy(
                    _tab.at[idx_v.at[pl.ds(q * CHUNK, CHUNK)]],
                    ring_v.at[par],
                    gsem,
                ).wait()
                pltpu.async_copy(
                    ring_v.at[par],
                    out_hbm.at[_f, pl.ds(b0 + q * CHUNK, CHUNK)],
                    wsem,
                )
                return carry

            lax.fori_loop(0, NCH, step_body, 0)
            for _ in range(2):
                pltpu.make_async_copy(
                    ring_v.at[0],
                    out_hbm.at[f, pl.ds(b0, CHUNK)],
                    wsem,
                ).wait()

    return k(tf_t, *tabs)


def kernel(t_features, tables):
    tf_t = t_features.astype(jnp.int32).T
    tabs = [tables[i] for i in range(N_FIELDS)]
    out3 = _sc_embed(tf_t, tabs)
    return out3.transpose(1, 0, 2).reshape(BATCH, N_FIELDS * EMBD)


# speedup vs baseline: 1.6716x; 1.6716x over previous
import functools

import jax
import jax.numpy as jnp
from jax import lax
from jax.experimental import pallas as pl
from jax.experimental.pallas import tpu as pltpu
from jax.experimental.pallas import tpu_sc as plsc

N_FIELDS = 26
VOCAB = 100000
EMBD = 32
BATCH = 16384

NUM_CORES = 2
NUM_SUBCORES = 16
NW = NUM_CORES * NUM_SUBCORES
FLAT = BATCH * N_FIELDS
PER_W = FLAT // NW
CHUNK = 128
NCH = PER_W // CHUNK
NBUF = 8
NGRP = NCH // NBUF


def _sc_embed(tf2d, tab_flat):
    mesh = plsc.VectorSubcoreMesh(core_axis_name="c", subcore_axis_name="s")

    @functools.partial(
        pl.kernel,
        mesh=mesh,
        out_type=jax.ShapeDtypeStruct((FLAT, EMBD), jnp.float32),
        compiler_params=pltpu.CompilerParams(use_tc_tiling_on_sc=False),
        scratch_types=[
            pltpu.VMEM((NCH, CHUNK), jnp.int32),
            pltpu.VMEM((2, NBUF, CHUNK, EMBD), jnp.float32),
            pltpu.SemaphoreType.DMA,
            pltpu.SemaphoreType.DMA,
        ],
    )
    def k(tf_hbm, tab_hbm, out_hbm, idx_v, rows_v, gsem, wsem):
        wid = lax.axis_index("s") * NUM_CORES + lax.axis_index("c")
        row0 = wid * NCH
        pltpu.sync_copy(tf_hbm.at[pl.ds(row0, NCH)], idx_v)

        iota = lax.broadcasted_iota(jnp.int32, (16,), 0)

        def off_body(r, carry):
            base = r * CHUNK
            for b in range(CHUNK // 16):
                p = (base + b * 16) + iota
                f = lax.rem(p, N_FIELDS)
                sl = pl.ds(b * 16, 16)
                idx_v[r, sl] = idx_v[r, sl] + f * VOCAB
            return carry

        lax.fori_loop(0, NCH, off_body, 0)

        out_base = wid * PER_W

        def body(g, carry):
            parity = lax.rem(g, 2)

            @pl.when(g >= 2)
            def _():
                for b in range(NBUF):
                    pltpu.make_async_copy(
                        rows_v.at[parity, b],
                        out_hbm.at[pl.ds(out_base, CHUNK)],
                        wsem,
                    ).wait()

            handles = []
            for b in range(NBUF):
                j = g * NBUF + b
                handles.append(
                    pltpu.async_copy(
                        tab_hbm.at[idx_v.at[j]],
                        rows_v.at[parity, b],
                        gsem,
                    )
                )
            for h in handles:
                h.wait()
            for b in range(NBUF):
                j = g * NBUF + b
                pltpu.async_copy(
                    rows_v.at[parity, b],
                    out_hbm.at[pl.ds(out_base + j * CHUNK, CHUNK)],
                    wsem,
                )
            return carry

        lax.fori_loop(0, NGRP, body, 0)

        for _ in range(2 * NBUF):
            pltpu.make_async_copy(
                rows_v.at[0, 0],
                out_hbm.at[pl.ds(out_base, CHUNK)],
                wsem,
            ).wait()

    return k(tf2d, tab_flat)


def kernel(t_features, tables):
    tf2d = t_features.astype(jnp.int32).reshape(NW * NCH, CHUNK)
    tab_flat = tables.reshape(N_FIELDS * VOCAB, EMBD)
    out = _sc_embed(tf2d, tab_flat)
    return out.reshape(BATCH, N_FIELDS * EMBD)
